# baseline (speedup 1.0000x reference)
import jax
import jax.numpy as jnp
from jax.experimental import pallas as pl
from jax.experimental.pallas import tpu as pltpu

_OUT_DIM = 128
_Q = 65536
_B = 4096


def _body(ptr_ref, k_ref, l_ref, q_ref, ql_ref, out_ref, outl_ref):
    del q_ref, ql_ref
    out_ref[...] = k_ref[...].T
    outl_ref[...] = l_ref[...]


def kernel(keys, labels, queue, queue_labels, queue_ptr):
    ptr = jnp.asarray(queue_ptr, jnp.int32)
    ptr_arr = jnp.reshape(ptr, (1,))
    labels2 = jnp.reshape(labels, (1, _B))
    qlabels2 = jnp.reshape(queue_labels, (1, _Q))

    grid_spec = pltpu.PrefetchScalarGridSpec(
        num_scalar_prefetch=1,
        grid=(1,),
        in_specs=[
            pl.BlockSpec((_B, _OUT_DIM), lambda j, p: (0, 0)),
            pl.BlockSpec((1, _B), lambda j, p: (0, 0)),
            pl.BlockSpec(memory_space=pl.ANY),
            pl.BlockSpec(memory_space=pl.ANY),
        ],
        out_specs=[
            pl.BlockSpec((_OUT_DIM, _B), lambda j, p: (0, p[0] // _B)),
            pl.BlockSpec((1, _B), lambda j, p: (0, p[0] // _B)),
        ],
    )

    new_queue, new_labels2 = pl.pallas_call(
        _body,
        grid_spec=grid_spec,
        out_shape=[
            jax.ShapeDtypeStruct((_OUT_DIM, _Q), jnp.float32),
            jax.ShapeDtypeStruct((1, _Q), jnp.int32),
        ],
        input_output_aliases={3: 0, 4: 1},
    )(ptr_arr, keys, labels2, queue, qlabels2)

    new_ptr = ((ptr + _B) % _Q).astype(jnp.int32)
    return new_queue, jnp.reshape(new_labels2, (_Q,)), new_ptr


# speedup vs baseline: 1.1404x; 1.1404x over previous
import jax
import jax.numpy as jnp
from jax.experimental import pallas as pl
from jax.experimental.pallas import tpu as pltpu

_OUT_DIM = 128
_Q = 65536
_B = 4096
_W = 16384
_NBLK = _Q // _W
_NHALF = _W // _B


def _body(ptr_ref, k_ref, l_ref, q_ref, ql_ref, out_ref, outl_ref):
    j = pl.program_id(0)
    ptr = ptr_ref[0]
    p0 = ptr // _W
    half = (ptr % _W) // _B

    out_ref[...] = q_ref[...]
    outl_ref[...] = ql_ref[...]

    @pl.when(j == p0)
    def _():
        for h in range(_NHALF):
            @pl.when(half == h)
            def _(h=h):
                out_ref[:, h * _B:(h + 1) * _B] = k_ref[...].T
                outl_ref[:, h * _B:(h + 1) * _B] = l_ref[...]


def kernel(keys, labels, queue, queue_labels, queue_ptr):
    ptr = jnp.asarray(queue_ptr, jnp.int32)
    ptr_arr = jnp.reshape(ptr, (1,))
    labels2 = jnp.reshape(labels, (1, _B))
    qlabels2 = jnp.reshape(queue_labels, (1, _Q))

    grid_spec = pltpu.PrefetchScalarGridSpec(
        num_scalar_prefetch=1,
        grid=(_NBLK,),
        in_specs=[
            pl.BlockSpec((_B, _OUT_DIM), lambda j, p: (0, 0)),
            pl.BlockSpec((1, _B), lambda j, p: (0, 0)),
            pl.BlockSpec((_OUT_DIM, _W), lambda j, p: (0, j)),
            pl.BlockSpec((1, _W), lambda j, p: (0, j)),
        ],
        out_specs=[
            pl.BlockSpec((_OUT_DIM, _W), lambda j, p: (0, j)),
            pl.BlockSpec((1, _W), lambda j, p: (0, j)),
        ],
    )

    new_queue, new_labels2 = pl.pallas_call(
        _body,
        grid_spec=grid_spec,
        out_shape=[
            jax.ShapeDtypeStruct((_OUT_DIM, _Q), jnp.float32),
            jax.ShapeDtypeStruct((1, _Q), jnp.int32),
        ],
    )(ptr_arr, keys, labels2, queue, qlabels2)

    new_ptr = ((ptr + _B) % _Q).astype(jnp.int32)
    return new_queue, jnp.reshape(new_labels2, (_Q,)), new_ptr
